# baseline (speedup 1.0000x reference)
import jax
import jax.numpy as jnp
from jax.experimental import pallas as pl
from jax.experimental.pallas import tpu as pltpu

_B = 32


def _tc_body(row_ref, col_ref, o_ref, pos_vmem):
    @pl.when(pl.program_id(0) == 0)
    def _compute_pos():
        row16 = row_ref[...]
        col16 = col_ref[...]
        hw = jax.lax.broadcasted_iota(jnp.int32, (16, 256), 1)
        lane = jax.lax.broadcasted_iota(jnp.int32, (16, 256), 0)
        sh = (lane == hw // 16).astype(jnp.float32)
        sw = (lane == hw % 16).astype(jnp.float32)
        dims = (((0,), (0,)), ((), ()))
        pos = jax.lax.dot_general(row16, sh, dims, preferred_element_type=jnp.float32)
        pos = pos + jax.lax.dot_general(col16, sw, dims, preferred_element_type=jnp.float32)
        pos_vmem[...] = pos

    o_ref[...] = pos_vmem[...][None]


def kernel(x, row_table, col_table):
    B, E, H, W = x.shape
    out3 = pl.pallas_call(
        _tc_body,
        grid=(B,),
        in_specs=[
            pl.BlockSpec((H, E), lambda b: (0, 0)),
            pl.BlockSpec((W, E), lambda b: (0, 0)),
        ],
        out_specs=pl.BlockSpec((1, E, H * W), lambda b: (b, 0, 0)),
        out_shape=jax.ShapeDtypeStruct((B, E, H * W), jnp.float32),
        scratch_shapes=[
            pltpu.VMEM((E, H * W), jnp.float32),
        ],
    )(row_table[:H], col_table[:W])
    return out3.reshape(B, E, H, W)


# speedup vs baseline: 1.2066x; 1.2066x over previous
import jax
import jax.numpy as jnp
from jax.experimental import pallas as pl
from jax.experimental.pallas import tpu as pltpu

_B = 32


_GRP = 4
_NSEM = 8


def _tc_body(row_ref, col_ref, o_hbm, pos_vmem, sems):
    row16 = row_ref[...]
    col16 = col_ref[...]
    hw = jax.lax.broadcasted_iota(jnp.int32, (16, 256), 1)
    lane = jax.lax.broadcasted_iota(jnp.int32, (16, 256), 0)
    sh = (lane == hw // 16).astype(jnp.float32)
    sw = (lane == hw % 16).astype(jnp.float32)
    dims = (((0,), (0,)), ((), ()))
    pos = jax.lax.dot_general(row16, sh, dims, preferred_element_type=jnp.float32)
    pos = pos + jax.lax.dot_general(col16, sw, dims, preferred_element_type=jnp.float32)
    for g in range(_GRP):
        pos_vmem[g] = pos
    ngrp = _B // _GRP
    for g in range(ngrp):
        pltpu.make_async_copy(
            pos_vmem, o_hbm.at[pl.ds(g * _GRP, _GRP)], sems.at[g % _NSEM]
        ).start()
    for g in range(ngrp):
        pltpu.make_async_copy(
            pos_vmem, o_hbm.at[pl.ds(g * _GRP, _GRP)], sems.at[g % _NSEM]
        ).wait()


def kernel(x, row_table, col_table):
    B, E, H, W = x.shape
    out3 = pl.pallas_call(
        _tc_body,
        in_specs=[
            pl.BlockSpec((H, E), lambda: (0, 0)),
            pl.BlockSpec((W, E), lambda: (0, 0)),
        ],
        out_specs=pl.BlockSpec(memory_space=pltpu.MemorySpace.HBM),
        out_shape=jax.ShapeDtypeStruct((B, E, H * W), jnp.float32),
        scratch_shapes=[
            pltpu.VMEM((_GRP, E, H * W), jnp.float32),
            pltpu.SemaphoreType.DMA((_NSEM,)),
        ],
    )(row_table[:H], col_table[:W])
    return out3.reshape(B, E, H, W)
